# baseline (speedup 1.0000x reference)
import functools

import jax
import jax.numpy as jnp
from jax.experimental import pallas as pl
from jax.experimental.pallas import tpu as pltpu

_SUB = 8


def _ceil_to(x, m):
    return ((x + m - 1) // m) * m


def _block_body(g_hbm, h0_ref, w_ref, b_ref, out_hbm, x_ref, ibuf, obuf,
                isem, osem, *, gk, nt, chunk):
    i = pl.program_id(0)
    n_chunks = nt // chunk
    total_chunks = pl.num_programs(0) * n_chunks
    n_slots = 2 if n_chunks % 2 == 0 else 1

    def _fetch(idx, slot):
        return pltpu.make_async_copy(
            g_hbm.at[pl.ds(idx * chunk, chunk)], ibuf.at[slot],
            isem.at[slot])

    def _drain(idx, slot):
        return pltpu.make_async_copy(
            obuf.at[slot], out_hbm.at[pl.ds(idx * chunk, chunk)],
            osem.at[slot])

    @pl.when(i == 0)
    def _seed():
        x_ref[:, gk:] = h0_ref[...]
        _fetch(0, 0).start()

    for c in range(n_chunks):
        slot = c % n_slots
        chunk_idx = i * n_chunks + c

        if n_slots == 2:
            @pl.when(chunk_idx + 1 < total_chunks)
            def _prefetch():
                _fetch(chunk_idx + 1, 1 - slot).start()

        _fetch(0, slot).wait()

        @pl.when(chunk_idx >= n_slots)
        def _reclaim():
            _drain(0, slot).wait()

        for s in range(chunk):
            x_ref[:, :gk] = ibuf[slot, s]
            h = jnp.maximum(
                jnp.dot(x_ref[...], w_ref[...],
                        preferred_element_type=jnp.float32) + b_ref[...],
                0.0,
            )
            x_ref[:, gk:] = h
            obuf[slot, s] = h

        _drain(chunk_idx, slot).start()

        if n_slots == 1:
            @pl.when(chunk_idx + 1 < total_chunks)
            def _fetch_seq():
                _fetch(chunk_idx + 1, 0).start()

    @pl.when(i == pl.num_programs(0) - 1)
    def _flush():
        for slot in range(n_slots):
            _drain(0, slot).wait()


def kernel(w_cat, b_cat, g_seq, hidden0):
    T, B, input_size = g_seq.shape
    hidden_size = hidden0.shape[1]
    h_pad = w_cat.shape[1]
    in_pad = w_cat.shape[0] - h_pad
    k_pad = in_pad + h_pad
    b_pad = _ceil_to(B, _SUB)

    nt = 1
    while nt < 16 and T % (nt * 2) == 0:
        nt *= 2
    chunk = nt // 2 if nt % 2 == 0 else nt

    g_p = g_seq.astype(jnp.float32)
    if (b_pad, in_pad) != (B, input_size):
        g_p = jnp.zeros((T, b_pad, in_pad), jnp.float32).at[
            :, :B, :input_size].set(g_p)
    h0_p = hidden0.astype(jnp.float32)
    if (b_pad, h_pad) != (B, hidden_size):
        h0_p = jnp.zeros((b_pad, h_pad), jnp.float32).at[
            :B, :hidden_size].set(h0_p)

    body = functools.partial(_block_body, gk=in_pad, nt=nt, chunk=chunk)

    h_seq = pl.pallas_call(
        body,
        out_shape=jax.ShapeDtypeStruct((T, b_pad, h_pad), jnp.float32),
        grid=(T // nt,),
        in_specs=[
            pl.BlockSpec(memory_space=pl.ANY),
            pl.BlockSpec((b_pad, h_pad), lambda i: (0, 0)),
            pl.BlockSpec((k_pad, h_pad), lambda i: (0, 0)),
            pl.BlockSpec((1, h_pad), lambda i: (0, 0)),
        ],
        out_specs=pl.BlockSpec(memory_space=pl.ANY),
        scratch_shapes=[
            pltpu.VMEM((b_pad, k_pad), jnp.float32),
            pltpu.VMEM((2, chunk, b_pad, in_pad), jnp.float32),
            pltpu.VMEM((2, chunk, b_pad, h_pad), jnp.float32),
            pltpu.SemaphoreType.DMA((2,)),
            pltpu.SemaphoreType.DMA((2,)),
        ],
        compiler_params=pltpu.CompilerParams(
            dimension_semantics=("arbitrary",)),
    )(g_p, h0_p, w_cat.astype(jnp.float32), b_cat.astype(jnp.float32))

    if (b_pad, h_pad) != (B, hidden_size):
        h_seq = h_seq[:, :B, :hidden_size]
    return h_seq


# speedup vs baseline: 1.0803x; 1.0803x over previous
import functools

import jax
import jax.numpy as jnp
from jax.experimental import pallas as pl
from jax.experimental.pallas import tpu as pltpu

_SUB = 8


def _ceil_to(x, m):
    return ((x + m - 1) // m) * m


def _block_body(g_ref, h0_ref, w_ref, b_ref, out_hbm, x_ref, obuf, osem,
                *, gk, nt, chunk):
    i = pl.program_id(0)
    n_chunks = nt // chunk
    n_slots = 2 if n_chunks % 2 == 0 else 1

    @pl.when(i == 0)
    def _seed():
        x_ref[:, gk:] = h0_ref[...]

    for c in range(n_chunks):
        slot = c % n_slots
        chunk_idx = i * n_chunks + c

        @pl.when(chunk_idx >= n_slots)
        def _reclaim():
            pltpu.make_async_copy(
                obuf.at[slot], out_hbm.at[pl.ds(0, chunk)], osem.at[slot]
            ).wait()

        for s in range(c * chunk, (c + 1) * chunk):
            x_ref[:, :gk] = g_ref[s]
            h = jnp.maximum(
                jnp.dot(x_ref[...], w_ref[...],
                        preferred_element_type=jnp.float32) + b_ref[...],
                0.0,
            )
            x_ref[:, gk:] = h
            obuf[slot, s - c * chunk] = h

        pltpu.make_async_copy(
            obuf.at[slot],
            out_hbm.at[pl.ds(chunk_idx * chunk, chunk)],
            osem.at[slot],
        ).start()

    @pl.when(i == pl.num_programs(0) - 1)
    def _flush():
        for slot in range(n_slots):
            pltpu.make_async_copy(
                obuf.at[slot], out_hbm.at[pl.ds(0, chunk)], osem.at[slot]
            ).wait()


def kernel(w_cat, b_cat, g_seq, hidden0):
    T, B, input_size = g_seq.shape
    hidden_size = hidden0.shape[1]
    h_pad = w_cat.shape[1]
    in_pad = w_cat.shape[0] - h_pad
    k_pad = in_pad + h_pad
    b_pad = _ceil_to(B, _SUB)

    nt = 1
    while nt < 16 and T % (nt * 2) == 0:
        nt *= 2
    chunk = nt
    if nt % 8 == 0:
        chunk = nt // 2
    elif nt % 2 == 0:
        chunk = nt // 2

    g_p = g_seq.astype(jnp.float32)
    if (b_pad, in_pad) != (B, input_size):
        g_p = jnp.zeros((T, b_pad, in_pad), jnp.float32).at[
            :, :B, :input_size].set(g_p)
    h0_p = hidden0.astype(jnp.float32)
    if (b_pad, h_pad) != (B, hidden_size):
        h0_p = jnp.zeros((b_pad, h_pad), jnp.float32).at[
            :B, :hidden_size].set(h0_p)

    body = functools.partial(_block_body, gk=in_pad, nt=nt, chunk=chunk)

    h_seq = pl.pallas_call(
        body,
        out_shape=jax.ShapeDtypeStruct((T, b_pad, h_pad), jnp.float32),
        grid=(T // nt,),
        in_specs=[
            pl.BlockSpec((nt, b_pad, in_pad), lambda i: (i, 0, 0)),
            pl.BlockSpec((b_pad, h_pad), lambda i: (0, 0)),
            pl.BlockSpec((k_pad, h_pad), lambda i: (0, 0)),
            pl.BlockSpec((1, h_pad), lambda i: (0, 0)),
        ],
        out_specs=pl.BlockSpec(memory_space=pl.ANY),
        scratch_shapes=[
            pltpu.VMEM((b_pad, k_pad), jnp.float32),
            pltpu.VMEM((2, chunk, b_pad, h_pad), jnp.float32),
            pltpu.SemaphoreType.DMA((2,)),
        ],
        compiler_params=pltpu.CompilerParams(
            dimension_semantics=("arbitrary",)),
    )(g_p, h0_p, w_cat.astype(jnp.float32), b_cat.astype(jnp.float32))

    if (b_pad, h_pad) != (B, hidden_size):
        h_seq = h_seq[:, :B, :hidden_size]
    return h_seq


# speedup vs baseline: 1.0820x; 1.0016x over previous
import functools

import jax
import jax.numpy as jnp
from jax.experimental import pallas as pl
from jax.experimental.pallas import tpu as pltpu

_SUB = 8


def _ceil_to(x, m):
    return ((x + m - 1) // m) * m


def _block_body(g_ref, h0_ref, w_ref, b_ref, out_hbm, x_ref, obuf, osem,
                *, gk, nt, chunk):
    i = pl.program_id(0)
    n_chunks = nt // chunk
    n_slots = 2 if n_chunks % 2 == 0 else 1

    @pl.when(i == 0)
    def _seed():
        x_ref[:, gk:] = h0_ref[...]

    for c in range(n_chunks):
        slot = c % n_slots
        chunk_idx = i * n_chunks + c

        @pl.when(chunk_idx >= n_slots)
        def _reclaim():
            pltpu.make_async_copy(
                obuf.at[slot], out_hbm.at[pl.ds(0, chunk)], osem.at[slot]
            ).wait()

        for s in range(c * chunk, (c + 1) * chunk):
            x_ref[:, :gk] = g_ref[s]
            h = jnp.maximum(
                jnp.dot(x_ref[...], w_ref[...],
                        preferred_element_type=jnp.float32) + b_ref[...],
                0.0,
            )
            x_ref[:, gk:] = h
            obuf[slot, s - c * chunk] = h

        pltpu.make_async_copy(
            obuf.at[slot],
            out_hbm.at[pl.ds(chunk_idx * chunk, chunk)],
            osem.at[slot],
        ).start()

    @pl.when(i == pl.num_programs(0) - 1)
    def _flush():
        for slot in range(n_slots):
            pltpu.make_async_copy(
                obuf.at[slot], out_hbm.at[pl.ds(0, chunk)], osem.at[slot]
            ).wait()


def kernel(w_cat, b_cat, g_seq, hidden0):
    T, B, input_size = g_seq.shape
    hidden_size = hidden0.shape[1]
    h_pad = w_cat.shape[1]
    in_pad = w_cat.shape[0] - h_pad
    k_pad = in_pad + h_pad
    b_pad = _ceil_to(B, _SUB)

    nt = 1
    while nt < 16 and T % (nt * 2) == 0:
        nt *= 2
    chunk = nt // 2 if nt % 2 == 0 else nt

    g_p = g_seq.astype(jnp.float32)
    if (b_pad, in_pad) != (B, input_size):
        g_p = jnp.zeros((T, b_pad, in_pad), jnp.float32).at[
            :, :B, :input_size].set(g_p)
    h0_p = hidden0.astype(jnp.float32)
    if (b_pad, h_pad) != (B, hidden_size):
        h0_p = jnp.zeros((b_pad, h_pad), jnp.float32).at[
            :B, :hidden_size].set(h0_p)

    body = functools.partial(_block_body, gk=in_pad, nt=nt, chunk=chunk)

    h_seq = pl.pallas_call(
        body,
        out_shape=jax.ShapeDtypeStruct((T, b_pad, h_pad), jnp.float32),
        grid=(T // nt,),
        in_specs=[
            pl.BlockSpec((nt, b_pad, in_pad), lambda i: (i, 0, 0)),
            pl.BlockSpec((b_pad, h_pad), lambda i: (0, 0)),
            pl.BlockSpec((k_pad, h_pad), lambda i: (0, 0)),
            pl.BlockSpec((1, h_pad), lambda i: (0, 0)),
        ],
        out_specs=pl.BlockSpec(memory_space=pl.ANY),
        scratch_shapes=[
            pltpu.VMEM((b_pad, k_pad), jnp.float32),
            pltpu.VMEM((2, chunk, b_pad, h_pad), jnp.float32),
            pltpu.SemaphoreType.DMA((2,)),
        ],
        compiler_params=pltpu.CompilerParams(
            dimension_semantics=("arbitrary",)),
    )(g_p, h0_p, w_cat.astype(jnp.float32), b_cat.astype(jnp.float32))

    if (b_pad, h_pad) != (B, hidden_size):
        h_seq = h_seq[:, :B, :hidden_size]
    return h_seq
